# baseline (speedup 1.0000x reference)
import functools

import jax
import jax.numpy as jnp
from jax import lax
from jax.experimental import pallas as pl
from jax.experimental.pallas import tpu as pltpu
from jax.experimental.pallas import tpu_sc as plsc


def _rowsum_body(wvt_ref, out_ref, *, inv_l, cols):
    x = wvt_ref[...]
    s = jnp.sum(x, axis=0) * inv_l
    out_ref[...] = s.reshape(cols // 128, 128)


def _rowsum(word_vectors, L):
    voc, d = word_vectors.shape
    wvt = word_vectors.T
    cols = 65536
    grid = pl.cdiv(voc, cols)
    out_rows = 8192
    assert grid * (cols // 128) <= out_rows
    return pl.pallas_call(
        functools.partial(_rowsum_body, inv_l=1.0 / float(L), cols=cols),
        grid=(grid,),
        in_specs=[pl.BlockSpec((d, cols), lambda i: (0, i))],
        out_specs=pl.BlockSpec((cols // 128, 128), lambda i: (i, 0)),
        out_shape=jax.ShapeDtypeStruct((out_rows, 128), jnp.float32),
    )(wvt)


def _make_sc_gather(bsz, L):
    info = plsc.get_sparse_core_info()
    nc, ns = info.num_cores, info.num_subcores
    nw = nc * ns
    n_idx = bsz * L
    assert n_idx % nw == 0 and bsz % (nw * 8) == 0
    per_s = bsz // nw
    mesh = plsc.VectorSubcoreMesh(core_axis_name="c", subcore_axis_name="s")

    @functools.partial(
        pl.kernel,
        out_type=jax.ShapeDtypeStruct((n_idx,), jnp.float32),
        mesh=mesh,
        scratch_types=[
            pltpu.VMEM((L * per_s,), jnp.int32),
            pltpu.VMEM((L * per_s,), jnp.float32),
            pltpu.SemaphoreType.DMA,
            pltpu.SemaphoreType.DMA,
            pltpu.SemaphoreType.DMA,
        ],
    )
    def gather_k(rowsum_hbm, idx_hbm, out_hbm, idx_v, val_v, sem_i, sem_g,
                 sem_o):
        wid = lax.axis_index("s") * nc + lax.axis_index("c")
        base = wid * per_s
        ld = [pltpu.async_copy(idx_hbm.at[pl.ds(l * bsz + base, per_s)],
                               idx_v.at[pl.ds(l * per_s, per_s)], sem_i)
              for l in range(L)]
        for h in ld:
            h.wait()
        pltpu.async_copy(rowsum_hbm.at[idx_v], val_v, sem_g).wait()
        st = [pltpu.async_copy(val_v.at[pl.ds(l * per_s, per_s)],
                               out_hbm.at[pl.ds(l * bsz + base, per_s)],
                               sem_o) for l in range(L)]
        for h in st:
            h.wait()

    return gather_k


def _finish_body(v_ref, w_ref, b_ref, out_ref, *, blk_b, L, qrows):
    p = pl.program_id(0)
    rows_per_l = v_ref.shape[0] // L
    acc = jnp.zeros((qrows, 128), jnp.float32)
    for l in range(L):
        acc = acc + v_ref[pl.ds(l * rows_per_l + qrows * p, qrows), :]
    ones = jnp.ones((1, w_ref.shape[1]), jnp.float32)
    wsum_row = lax.dot_general(ones, w_ref[...], (((1,), (1,)), ((), ())),
                               preferred_element_type=jnp.float32)
    parts = []
    for q in range(qrows):
        s_row = acc[q:q + 1, :]
        parts.append(lax.dot_general(
            s_row, wsum_row, (((0,), (0,)), ((), ())),
            preferred_element_type=jnp.float32) + b_ref[...])
    out_ref[...] = jnp.concatenate(parts, axis=0)


def _finish(vals, bsz, L, W, b):
    n_labels, d = W.shape
    blk_b = 1024
    qrows = blk_b // 128
    v2d = vals.reshape(bsz * L // 128, 128)
    return pl.pallas_call(
        functools.partial(_finish_body, blk_b=blk_b, L=L, qrows=qrows),
        grid=(bsz // blk_b,),
        in_specs=[
            pl.BlockSpec(v2d.shape, lambda i: (0, 0)),
            pl.BlockSpec((n_labels, d), lambda i: (0, 0)),
            pl.BlockSpec((1, n_labels), lambda i: (0, 0)),
        ],
        out_specs=pl.BlockSpec((blk_b, n_labels), lambda i: (i, 0)),
        out_shape=jax.ShapeDtypeStruct((bsz, n_labels), jnp.float32),
    )(v2d, W, b.reshape(1, n_labels))


def kernel(sentences, word_vectors, W, b):
    bsz, L = sentences.shape
    rowsum = _rowsum(word_vectors, L).reshape(-1)
    idx = sentences.T.reshape(-1).astype(jnp.int32)
    vals = _make_sc_gather(bsz, L)(rowsum, idx)
    return _finish(vals, bsz, L, W, b)


# speedup vs baseline: 1.0155x; 1.0155x over previous
import functools

import jax
import jax.numpy as jnp
from jax import lax
from jax.experimental import pallas as pl
from jax.experimental.pallas import tpu as pltpu
from jax.experimental.pallas import tpu_sc as plsc


def _rowsum_body(wvt_ref, out_ref, *, inv_l, cols):
    x = wvt_ref[...]
    s = jnp.sum(x, axis=0) * inv_l
    out_ref[...] = s.reshape(cols // 128, 128)


def _rowsum(word_vectors, L):
    voc, d = word_vectors.shape
    wvt = word_vectors.T
    cols = 32768
    grid = pl.cdiv(voc, cols)
    out_rows = 8192
    assert grid * (cols // 128) <= out_rows
    return pl.pallas_call(
        functools.partial(_rowsum_body, inv_l=1.0 / float(L), cols=cols),
        grid=(grid,),
        in_specs=[pl.BlockSpec((d, cols), lambda i: (0, i))],
        out_specs=pl.BlockSpec((cols // 128, 128), lambda i: (i, 0)),
        out_shape=jax.ShapeDtypeStruct((out_rows, 128), jnp.float32),
    )(wvt)


def _make_sc_gather(bsz, L):
    info = plsc.get_sparse_core_info()
    nc, ns = info.num_cores, info.num_subcores
    nw = nc * ns
    n_idx = bsz * L
    assert n_idx % nw == 0 and bsz % (nw * 8) == 0
    per_s = bsz // nw
    mesh = plsc.VectorSubcoreMesh(core_axis_name="c", subcore_axis_name="s")

    @functools.partial(
        pl.kernel,
        out_type=jax.ShapeDtypeStruct((n_idx,), jnp.float32),
        mesh=mesh,
        scratch_types=[
            pltpu.VMEM((L * per_s,), jnp.int32),
            pltpu.VMEM((L * per_s,), jnp.float32),
            pltpu.SemaphoreType.DMA,
            pltpu.SemaphoreType.DMA,
            pltpu.SemaphoreType.DMA,
        ],
    )
    def gather_k(rowsum_hbm, idx_hbm, out_hbm, idx_v, val_v, sem_i, sem_g,
                 sem_o):
        wid = lax.axis_index("s") * nc + lax.axis_index("c")
        base = wid * per_s
        ld = [pltpu.async_copy(idx_hbm.at[pl.ds(l * bsz + base, per_s)],
                               idx_v.at[pl.ds(l * per_s, per_s)], sem_i)
              for l in range(L)]
        for h in ld:
            h.wait()
        pltpu.async_copy(rowsum_hbm.at[idx_v], val_v, sem_g).wait()
        st = [pltpu.async_copy(val_v.at[pl.ds(l * per_s, per_s)],
                               out_hbm.at[pl.ds(l * bsz + base, per_s)],
                               sem_o) for l in range(L)]
        for h in st:
            h.wait()

    return gather_k


def _finish_body(v_ref, w_ref, b_ref, out_ref, *, blk_b, L, qrows):
    p = pl.program_id(0)
    rows_per_l = v_ref.shape[0] // L
    acc = jnp.zeros((qrows, 128), jnp.float32)
    for l in range(L):
        acc = acc + v_ref[pl.ds(l * rows_per_l + qrows * p, qrows), :]
    ones = jnp.ones((1, w_ref.shape[1]), jnp.float32)
    wsum_row = lax.dot_general(ones, w_ref[...], (((1,), (1,)), ((), ())),
                               preferred_element_type=jnp.float32)
    parts = []
    for q in range(qrows):
        s_row = acc[q:q + 1, :]
        parts.append(lax.dot_general(
            s_row, wsum_row, (((0,), (0,)), ((), ())),
            preferred_element_type=jnp.float32) + b_ref[...])
    out_ref[...] = jnp.concatenate(parts, axis=0)


def _finish(vals, bsz, L, W, b):
    n_labels, d = W.shape
    blk_b = 1024
    qrows = blk_b // 128
    v2d = vals.reshape(bsz * L // 128, 128)
    return pl.pallas_call(
        functools.partial(_finish_body, blk_b=blk_b, L=L, qrows=qrows),
        grid=(bsz // blk_b,),
        in_specs=[
            pl.BlockSpec(v2d.shape, lambda i: (0, 0)),
            pl.BlockSpec((n_labels, d), lambda i: (0, 0)),
            pl.BlockSpec((1, n_labels), lambda i: (0, 0)),
        ],
        out_specs=pl.BlockSpec((blk_b, n_labels), lambda i: (i, 0)),
        out_shape=jax.ShapeDtypeStruct((bsz, n_labels), jnp.float32),
    )(v2d, W, b.reshape(1, n_labels))


def kernel(sentences, word_vectors, W, b):
    bsz, L = sentences.shape
    rowsum = _rowsum(word_vectors, L).reshape(-1)
    idx = sentences.T.reshape(-1).astype(jnp.int32)
    vals = _make_sc_gather(bsz, L)(rowsum, idx)
    return _finish(vals, bsz, L, W, b)
